# Initial kernel scaffold; baseline (speedup 1.0000x reference)
import functools

import jax
import jax.numpy as jnp
from jax import lax
from jax.experimental import pallas as pl
from jax.experimental.pallas import tpu as pltpu
from jax.experimental.pallas import tpu_sc as plsc

N_NODES = 100000
EMB_DIM = 128
CHUNK = 128
NC, NS = 2, 16
NW = NC * NS
NCHUNKS = (N_NODES + CHUNK - 1) // CHUNK
TAIL_ROWS = N_NODES - (NCHUNKS - 1) * CHUNK
MAXC = (NCHUNKS + NW - 1) // NW
N_EXTRA = NCHUNKS - NW * (MAXC - 1)
IDX_ROWS = NW * MAXC
NBUF = 3


@functools.partial(
    pl.kernel,
    out_type=jax.ShapeDtypeStruct((N_NODES, EMB_DIM), jnp.float32),
    mesh=plsc.VectorSubcoreMesh(
        core_axis_name="c", subcore_axis_name="s", num_cores=NC, num_subcores=NS
    ),
    scratch_types=[
        pltpu.VMEM((MAXC, CHUNK), jnp.int32),
        pltpu.VMEM((NBUF, CHUNK, EMB_DIM), jnp.float32),
        pltpu.SemaphoreType.DMA,
        pltpu.SemaphoreType.DMA,
    ],
)
def _sc_gather(idx_hbm, table_hbm, out_hbm, idx_v, buf_v, gsem, ssem):
    c = lax.axis_index("c")
    s = lax.axis_index("s")
    w = s * NC + c
    start = (MAXC - 1) * w + jnp.minimum(w, N_EXTRA)

    pltpu.sync_copy(idx_hbm.at[pl.ds(start, MAXC)], idx_v)

    def g_desc(j):
        return pltpu.make_async_copy(
            table_hbm.at[idx_v.at[j]], buf_v.at[j % NBUF], gsem
        )

    def s_full(j):
        return pltpu.make_async_copy(
            buf_v.at[j % NBUF],
            out_hbm.at[pl.ds((start + j) * CHUNK, CHUNK)],
            ssem,
        )

    def s_tail(j):
        return pltpu.make_async_copy(
            buf_v.at[j % NBUF, pl.ds(0, TAIL_ROWS)],
            out_hbm.at[pl.ds((NCHUNKS - 1) * CHUNK, TAIL_ROWS)],
            ssem,
        )

    def when_valid(j, fn):
        if j < MAXC - 1:
            fn()
        else:
            pl.when(w < N_EXTRA)(fn)

    def start_store(j):
        if j == MAXC - 2:
            pl.when(w == NW - 1)(lambda: s_tail(j).start())
            pl.when(w != NW - 1)(lambda: s_full(j).start())
        else:
            when_valid(j, lambda: s_full(j).start())

    def wait_store(j):
        if j == MAXC - 2:
            pl.when(w == NW - 1)(lambda: s_tail(j).wait())
            pl.when(w != NW - 1)(lambda: s_full(j).wait())
        else:
            when_valid(j, lambda: s_full(j).wait())

    g_desc(0).start()
    g_desc(1).start()

    for j in range(MAXC):
        when_valid(j, lambda j=j: g_desc(j).wait())
        start_store(j)
        jn = j + 2
        if jn < MAXC:
            if j >= 1:
                wait_store(j - 1)
            when_valid(jn, lambda jn=jn: g_desc(jn).start())

    for j in range(max(0, MAXC - 3), MAXC):
        wait_store(j)


# speedup vs baseline: 1.5337x; 1.5337x over previous
import functools

import jax
import jax.numpy as jnp
from jax import lax
from jax.experimental import pallas as pl
from jax.experimental.pallas import tpu as pltpu
from jax.experimental.pallas import tpu_sc as plsc

N_NODES = 100000
EMB_DIM = 128
CHUNK = 128
NC, NS = 2, 16
NW = NC * NS
NCHUNKS = (N_NODES + CHUNK - 1) // CHUNK
TAIL_ROWS = N_NODES - (NCHUNKS - 1) * CHUNK
MAXC = (NCHUNKS + NW - 1) // NW
TAIL_SLOT = NCHUNKS - 1 - MAXC * (NW - 1)
IDX_SLOTS = 32
NBUF = 3


@functools.partial(
    pl.kernel,
    out_type=jax.ShapeDtypeStruct((N_NODES, EMB_DIM), jnp.float32),
    mesh=plsc.VectorSubcoreMesh(
        core_axis_name="c", subcore_axis_name="s", num_cores=NC, num_subcores=NS
    ),
    scratch_types=[
        pltpu.VMEM((IDX_SLOTS, CHUNK), jnp.int32),
        pltpu.VMEM((NBUF, CHUNK, EMB_DIM), jnp.float32),
        pltpu.SemaphoreType.DMA,
        pltpu.SemaphoreType.DMA,
    ],
)
def _sc_gather(idx_hbm, table_hbm, out_hbm, idx_v, buf_v, gsem, ssem):
    c = lax.axis_index("c")
    s = lax.axis_index("s")
    w = s * NC + c

    pltpu.sync_copy(idx_hbm.at[w], idx_v)

    def g_desc(j):
        return pltpu.make_async_copy(
            table_hbm.at[idx_v.at[j]], buf_v.at[j % NBUF], gsem
        )

    def s_full(j):
        return pltpu.make_async_copy(
            buf_v.at[j % NBUF],
            out_hbm.at[pl.ds((MAXC * w + j) * CHUNK, CHUNK)],
            ssem,
        )

    def s_tail(j):
        return pltpu.make_async_copy(
            buf_v.at[j % NBUF, pl.ds(0, TAIL_ROWS)],
            out_hbm.at[pl.ds((NCHUNKS - 1) * CHUNK, TAIL_ROWS)],
            ssem,
        )

    def when_valid(j, fn):
        if j <= TAIL_SLOT:
            fn()
        else:
            pl.when(w < NW - 1)(fn)

    def start_store(j):
        if j == TAIL_SLOT:
            pl.when(w == NW - 1)(lambda: s_tail(j).start())
            pl.when(w != NW - 1)(lambda: s_full(j).start())
        else:
            when_valid(j, lambda: s_full(j).start())

    def wait_store(j):
        if j == TAIL_SLOT:
            pl.when(w == NW - 1)(lambda: s_tail(j).wait())
            pl.when(w != NW - 1)(lambda: s_full(j).wait())
        else:
            when_valid(j, lambda: s_full(j).wait())

    g_desc(0).start()
    g_desc(1).start()

    for j in range(MAXC):
        when_valid(j, lambda j=j: g_desc(j).wait())
        start_store(j)
        jn = j + 2
        if jn < MAXC:
            if j >= 1:
                wait_store(j - 1)
            when_valid(jn, lambda jn=jn: g_desc(jn).start())

    for j in range(max(0, MAXC - 3), MAXC):
        wait_store(j)


def kernel(node_feature, atom_type_embedding):
    idx = node_feature[:, 0]
    idx = jnp.pad(idx, (0, NW * MAXC * CHUNK - N_NODES))
    idx = idx.reshape(NW, MAXC, CHUNK)
    idx = jnp.pad(idx, ((0, 0), (0, IDX_SLOTS - MAXC), (0, 0)))
    return _sc_gather(idx, atom_type_embedding)
